# baseline (speedup 1.0000x reference)
import jax
import jax.numpy as jnp
from jax import lax
from jax.experimental import pallas as pl
from jax.experimental.pallas import tpu as pltpu
from jax.experimental.pallas import tpu_sc as plsc

N = 10000
D = 128
H = 128
E = 320000

N_PAD = 10240
DUMP = N
NC, NS = 2, 16
NW = NC * NS
CHUNK = 128
CPW = 80
EPW = CPW * CHUNK
E_PAD = EPW * NW
RPS = N_PAD // NS

BLK = 256
GRID = N_PAD // BLK
LBLK = 400
LGRID = N // LBLK


def _sc_mesh():
    return plsc.VectorSubcoreMesh(
        core_axis_name="c", subcore_axis_name="s", num_cores=NC, num_subcores=NS)


def _deg_body(dst_hbm, zeros_hbm, out_hbm, didx, ones_v, acc):
    c = lax.axis_index("c")
    s = lax.axis_index("s")
    wid = c * NS + s

    def put1(i, carry):
        ones_v[i, :] = jnp.ones((16,), jnp.float32)
        return carry

    lax.fori_loop(0, CHUNK, put1, 0)
    pltpu.sync_copy(dst_hbm.at[wid], didx)
    pltpu.sync_copy(zeros_hbm, acc.at[pl.ds(s * RPS, RPS), :])
    plsc.subcore_barrier()

    def step(k, carry):
        pltpu.sync_copy(ones_v, acc.at[didx.at[k]], add=True)
        return carry

    lax.fori_loop(0, CPW, step, 0)
    plsc.subcore_barrier()
    pltpu.sync_copy(acc.at[pl.ds(s * RPS, RPS), :],
                    out_hbm.at[c, pl.ds(s * RPS, RPS), :])


def _deg_kernel(*args):
    return pl.kernel(
        _deg_body,
        out_type=jax.ShapeDtypeStruct((NC, N_PAD, 16), jnp.float32),
        mesh=_sc_mesh(),
        scratch_types=[
            pltpu.VMEM((CPW, CHUNK), jnp.int32),
            pltpu.VMEM((CHUNK, 16), jnp.float32),
            pltpu.VMEM_SHARED((N_PAD, 16), jnp.float32),
        ],
    )(*args)


def _agg_body(table_hbm, src_hbm, dst_hbm, zeros_hbm, out_hbm,
              didx, sbuf0, sbuf1, rows0, rows1, acc,
              semi0, semi1, semr0, semr1):
    c = lax.axis_index("c")
    s = lax.axis_index("s")
    wid = c * NS + s
    base = wid * EPW
    pltpu.sync_copy(dst_hbm.at[wid], didx)
    pltpu.sync_copy(zeros_hbm, acc.at[pl.ds(s * RPS, RPS), :])
    plsc.subcore_barrier()

    pltpu.sync_copy(src_hbm.at[pl.ds(base, CHUNK)], sbuf0)
    pltpu.async_copy(table_hbm.at[sbuf0], rows0, semr0)
    pltpu.async_copy(src_hbm.at[pl.ds(base + CHUNK, CHUNK)], sbuf1, semi1)

    def step(j, carry):
        a = 2 * j
        b = a + 1
        pltpu.make_async_copy(src_hbm.at[pl.ds(base + b * CHUNK, CHUNK)],
                              sbuf1, semi1).wait()
        pltpu.async_copy(table_hbm.at[sbuf1], rows1, semr1)
        pltpu.make_async_copy(table_hbm.at[sbuf0], rows0, semr0).wait()
        pltpu.sync_copy(rows0, acc.at[didx.at[a]], add=True)

        @pl.when(b + 1 < CPW)
        def _prep_a():
            pltpu.async_copy(src_hbm.at[pl.ds(base + (b + 1) * CHUNK, CHUNK)],
                             sbuf0, semi0)

        pltpu.make_async_copy(table_hbm.at[sbuf1], rows1, semr1).wait()
        pltpu.sync_copy(rows1, acc.at[didx.at[b]], add=True)

        @pl.when(b + 1 < CPW)
        def _gather_a():
            pltpu.make_async_copy(src_hbm.at[pl.ds(base + (b + 1) * CHUNK, CHUNK)],
                                  sbuf0, semi0).wait()
            pltpu.async_copy(table_hbm.at[sbuf0], rows0, semr0)

        @pl.when(b + 2 < CPW)
        def _prep_b():
            pltpu.async_copy(src_hbm.at[pl.ds(base + (b + 2) * CHUNK, CHUNK)],
                             sbuf1, semi1)

        return carry

    lax.fori_loop(0, CPW // 2, step, 0)
    plsc.subcore_barrier()
    pltpu.sync_copy(acc.at[pl.ds(s * RPS, RPS), :],
                    out_hbm.at[c, pl.ds(s * RPS, RPS), :])


def _agg_kernel(*args):
    return pl.kernel(
        _agg_body,
        out_type=jax.ShapeDtypeStruct((NC, N_PAD, D), jnp.float32),
        mesh=_sc_mesh(),
        scratch_types=[
            pltpu.VMEM((CPW, CHUNK), jnp.int32),
            pltpu.VMEM((CHUNK,), jnp.int32),
            pltpu.VMEM((CHUNK,), jnp.int32),
            pltpu.VMEM((CHUNK, D), jnp.float32),
            pltpu.VMEM((CHUNK, D), jnp.float32),
            pltpu.VMEM_SHARED((N_PAD, D), jnp.float32),
            pltpu.SemaphoreType.DMA,
            pltpu.SemaphoreType.DMA,
            pltpu.SemaphoreType.DMA,
            pltpu.SemaphoreType.DMA,
        ],
    )(*args)


def _dinv(d0_ref, d1_ref):
    deg = d0_ref[:, 0:1] + d1_ref[:, 0:1] + 1.0
    return lax.rsqrt(deg)


def _h1_body(x_ref, w1_ref, d0_ref, d1_ref, o_ref):
    o_ref[...] = jnp.dot(x_ref[...], w1_ref[...],
                         preferred_element_type=jnp.float32) * _dinv(d0_ref, d1_ref)


def _h2_body(p0_ref, p1_ref, h1_ref, d0_ref, d1_ref, b1_ref, w2_ref, o_ref):
    dinv = _dinv(d0_ref, d1_ref)
    agg = p0_ref[...] + p1_ref[...] + h1_ref[...]
    t = jnp.maximum(agg * dinv + b1_ref[...], 0.0)
    h2 = jnp.dot(t, w2_ref[...], preferred_element_type=jnp.float32) * dinv
    row = pl.program_id(0) * BLK + lax.broadcasted_iota(jnp.int32, (BLK, 1), 0)
    o_ref[...] = jnp.where(row < N, h2, 0.0)


def _xg_body(q0_ref, q1_ref, h2_ref, d0_ref, d1_ref, b2_ref, wih_ref,
             bih_ref, bhh_ref, o_ref):
    dinv = _dinv(d0_ref, d1_ref)
    agg = q0_ref[...] + q1_ref[...] + h2_ref[...]
    t = jnp.maximum(agg * dinv + b2_ref[...], 0.0)
    o_ref[...] = (jnp.dot(t, wih_ref[...], preferred_element_type=jnp.float32)
                  + bih_ref[...] + bhh_ref[...])


def _lstm_body(xg_ref, whh_ref, wc_ref, bc_ref, o_ref, h_s, c_s):
    @pl.when(pl.program_id(0) == 0)
    def _init():
        h_s[...] = jnp.zeros_like(h_s)
        c_s[...] = jnp.zeros_like(c_s)

    def step(t, carry):
        h, c = carry
        gates = xg_ref[pl.ds(t, 1), :] + jnp.dot(
            h.astype(jnp.bfloat16), whh_ref[...],
            preferred_element_type=jnp.float32)
        i_g = jax.nn.sigmoid(gates[:, 0:H])
        f_g = jax.nn.sigmoid(gates[:, H:2 * H])
        g_g = jnp.tanh(gates[:, 2 * H:3 * H])
        o_g = jax.nn.sigmoid(gates[:, 3 * H:4 * H])
        c2 = f_g * c + i_g * g_g
        h2 = o_g * jnp.tanh(c2)
        return (h2, c2)

    h, c = lax.fori_loop(0, LBLK, step, (h_s[...], c_s[...]), unroll=4)
    h_s[...] = h
    c_s[...] = c

    @pl.when(pl.program_id(0) == LGRID - 1)
    def _final():
        o_ref[...] = jnp.dot(h, wc_ref[...],
                             preferred_element_type=jnp.float32) + bc_ref[...]


def _full(shape):
    return pl.BlockSpec(shape, lambda i: (0,) * len(shape))


def _rows(shape):
    return pl.BlockSpec(shape, lambda i: (i,) + (0,) * (len(shape) - 1))


_h1_call = pl.pallas_call(
    _h1_body,
    grid=(GRID,),
    in_specs=[_rows((BLK, D)), _full((D, H)), _rows((BLK, 16)), _rows((BLK, 16))],
    out_specs=_rows((BLK, H)),
    out_shape=jax.ShapeDtypeStruct((N_PAD, H), jnp.float32),
)

_h2_call = pl.pallas_call(
    _h2_body,
    grid=(GRID,),
    in_specs=[_rows((BLK, H)), _rows((BLK, H)), _rows((BLK, H)),
              _rows((BLK, 16)), _rows((BLK, 16)), _full((1, H)), _full((H, H))],
    out_specs=_rows((BLK, H)),
    out_shape=jax.ShapeDtypeStruct((N_PAD, H), jnp.float32),
)

_xg_call = pl.pallas_call(
    _xg_body,
    grid=(GRID,),
    in_specs=[_rows((BLK, H)), _rows((BLK, H)), _rows((BLK, H)),
              _rows((BLK, 16)), _rows((BLK, 16)), _full((1, H)),
              _full((H, 4 * H)), _full((1, 4 * H)), _full((1, 4 * H))],
    out_specs=_rows((BLK, 4 * H)),
    out_shape=jax.ShapeDtypeStruct((N_PAD, 4 * H), jnp.float32),
)

_lstm_call = pl.pallas_call(
    _lstm_body,
    grid=(LGRID,),
    in_specs=[_rows((LBLK, 4 * H)), _full((H, 4 * H)), _full((H, 128)),
              _full((1, 128))],
    out_specs=_full((1, 128)),
    out_shape=jax.ShapeDtypeStruct((1, 128), jnp.float32),
    scratch_shapes=[pltpu.VMEM((1, H), jnp.float32),
                    pltpu.VMEM((1, H), jnp.float32)],
)


def kernel(x, edge_index, timestamp, W1, b1, W2, b2, W_ih, W_hh, b_ih, b_hh,
           Wc, bc):
    pad = jnp.full((E_PAD - E,), DUMP, jnp.int32)
    src = jnp.concatenate([edge_index[0], pad])
    dst = jnp.concatenate([edge_index[1], pad]).reshape(NW, CPW, CHUNK)
    x_pad = jnp.pad(x, ((0, N_PAD - N), (0, 0)))
    z16 = jnp.zeros((RPS, 16), jnp.float32)
    z128 = jnp.zeros((RPS, D), jnp.float32)
    b1r = b1.reshape(1, H)
    b2r = b2.reshape(1, H)
    bihr = b_ih.reshape(1, 4 * H)
    bhhr = b_hh.reshape(1, 4 * H)
    whh_bf = W_hh.astype(jnp.bfloat16)
    wc_pad = jnp.pad(Wc, ((0, 0), (0, 128 - Wc.shape[1])))
    bc_pad = jnp.pad(bc, (0, 128 - bc.shape[0])).reshape(1, 128)

    degs = _deg_kernel(dst, z16)
    d0, d1 = degs[0], degs[1]
    h1p = _h1_call(x_pad, W1, d0, d1)
    agg1 = _agg_kernel(h1p, src, dst, z128)
    h2p = _h2_call(agg1[0], agg1[1], h1p, d0, d1, b1r, W2)
    agg2 = _agg_kernel(h2p, src, dst, z128)
    xg = _xg_call(agg2[0], agg2[1], h2p, d0, d1, b2r, W_ih, bihr, bhhr)
    out = _lstm_call(xg, whh_bf, wc_pad, bc_pad)
    return out[:, :bc.shape[0]]


# speedup vs baseline: 1.0843x; 1.0843x over previous
import jax
import jax.numpy as jnp
from jax import lax
from jax.experimental import pallas as pl
from jax.experimental.pallas import tpu as pltpu
from jax.experimental.pallas import tpu_sc as plsc

N = 10000
D = 128
H = 128
E = 320000

N_PAD = 10240
DUMP = N
NC, NS = 2, 16
NW = NC * NS
CHUNK = 128
CPW = 80
EPW = CPW * CHUNK
E_PAD = EPW * NW
RPS = N_PAD // NS

BLK = 256
GRID = N_PAD // BLK
LBLK = 400
LGRID = N // LBLK


def _sc_mesh():
    return plsc.VectorSubcoreMesh(
        core_axis_name="c", subcore_axis_name="s", num_cores=NC, num_subcores=NS)


def _deg_body(dst_hbm, zeros_hbm, out_hbm, didx, ones_v, acc):
    c = lax.axis_index("c")
    s = lax.axis_index("s")
    wid = c * NS + s

    def put1(i, carry):
        ones_v[i, :] = jnp.ones((16,), jnp.float32)
        return carry

    lax.fori_loop(0, CHUNK, put1, 0)
    pltpu.sync_copy(dst_hbm.at[wid], didx)
    pltpu.sync_copy(zeros_hbm, acc.at[pl.ds(s * RPS, RPS), :])
    plsc.subcore_barrier()

    def step(k, carry):
        pltpu.sync_copy(ones_v, acc.at[didx.at[k]], add=True)
        return carry

    lax.fori_loop(0, CPW, step, 0)
    plsc.subcore_barrier()
    pltpu.sync_copy(acc.at[pl.ds(s * RPS, RPS), :],
                    out_hbm.at[c, pl.ds(s * RPS, RPS), :])


def _deg_kernel(*args):
    return pl.kernel(
        _deg_body,
        out_type=jax.ShapeDtypeStruct((NC, N_PAD, 16), jnp.float32),
        mesh=_sc_mesh(),
        scratch_types=[
            pltpu.VMEM((CPW, CHUNK), jnp.int32),
            pltpu.VMEM((CHUNK, 16), jnp.float32),
            pltpu.VMEM_SHARED((N_PAD, 16), jnp.float32),
        ],
    )(*args)


def _agg_body(table_hbm, src_hbm, dst_hbm, zeros_hbm, out_hbm,
              didx, sbuf0, sbuf1, rows0, rows1, acc,
              semi0, semi1, semr0, semr1):
    c = lax.axis_index("c")
    s = lax.axis_index("s")
    wid = c * NS + s
    base = wid * EPW
    pltpu.sync_copy(dst_hbm.at[wid], didx)
    pltpu.sync_copy(zeros_hbm, acc.at[pl.ds(s * RPS, RPS), :])
    plsc.subcore_barrier()

    pltpu.sync_copy(src_hbm.at[pl.ds(base, CHUNK)], sbuf0)
    pltpu.async_copy(table_hbm.at[sbuf0], rows0, semr0)
    pltpu.async_copy(src_hbm.at[pl.ds(base + CHUNK, CHUNK)], sbuf1, semi1)

    def step(j, carry):
        a = 2 * j
        b = a + 1
        pltpu.make_async_copy(src_hbm.at[pl.ds(base + b * CHUNK, CHUNK)],
                              sbuf1, semi1).wait()
        pltpu.async_copy(table_hbm.at[sbuf1], rows1, semr1)
        pltpu.make_async_copy(table_hbm.at[sbuf0], rows0, semr0).wait()
        pltpu.sync_copy(rows0, acc.at[didx.at[a]], add=True)

        @pl.when(b + 1 < CPW)
        def _prep_a():
            pltpu.async_copy(src_hbm.at[pl.ds(base + (b + 1) * CHUNK, CHUNK)],
                             sbuf0, semi0)

        pltpu.make_async_copy(table_hbm.at[sbuf1], rows1, semr1).wait()
        pltpu.sync_copy(rows1, acc.at[didx.at[b]], add=True)

        @pl.when(b + 1 < CPW)
        def _gather_a():
            pltpu.make_async_copy(src_hbm.at[pl.ds(base + (b + 1) * CHUNK, CHUNK)],
                                  sbuf0, semi0).wait()
            pltpu.async_copy(table_hbm.at[sbuf0], rows0, semr0)

        @pl.when(b + 2 < CPW)
        def _prep_b():
            pltpu.async_copy(src_hbm.at[pl.ds(base + (b + 2) * CHUNK, CHUNK)],
                             sbuf1, semi1)

        return carry

    lax.fori_loop(0, CPW // 2, step, 0)
    plsc.subcore_barrier()
    pltpu.sync_copy(acc.at[pl.ds(s * RPS, RPS), :],
                    out_hbm.at[c, pl.ds(s * RPS, RPS), :])


def _agg_kernel(*args):
    return pl.kernel(
        _agg_body,
        out_type=jax.ShapeDtypeStruct((NC, N_PAD, D), jnp.float32),
        mesh=_sc_mesh(),
        scratch_types=[
            pltpu.VMEM((CPW, CHUNK), jnp.int32),
            pltpu.VMEM((CHUNK,), jnp.int32),
            pltpu.VMEM((CHUNK,), jnp.int32),
            pltpu.VMEM((CHUNK, D), jnp.float32),
            pltpu.VMEM((CHUNK, D), jnp.float32),
            pltpu.VMEM_SHARED((N_PAD, D), jnp.float32),
            pltpu.SemaphoreType.DMA,
            pltpu.SemaphoreType.DMA,
            pltpu.SemaphoreType.DMA,
            pltpu.SemaphoreType.DMA,
        ],
    )(*args)


def _dinv(d0_ref, d1_ref):
    deg = d0_ref[:, 0:1] + d1_ref[:, 0:1] + 1.0
    return lax.rsqrt(deg)


def _h1_body(x_ref, w1_ref, d0_ref, d1_ref, o_ref):
    o_ref[...] = jnp.dot(x_ref[...], w1_ref[...],
                         preferred_element_type=jnp.float32) * _dinv(d0_ref, d1_ref)


def _h2_body(p0_ref, p1_ref, h1_ref, d0_ref, d1_ref, b1_ref, w2_ref, o_ref):
    dinv = _dinv(d0_ref, d1_ref)
    agg = p0_ref[...] + p1_ref[...] + h1_ref[...]
    t = jnp.maximum(agg * dinv + b1_ref[...], 0.0)
    h2 = jnp.dot(t, w2_ref[...], preferred_element_type=jnp.float32) * dinv
    row = pl.program_id(0) * BLK + lax.broadcasted_iota(jnp.int32, (BLK, 1), 0)
    o_ref[...] = jnp.where(row < N, h2, 0.0)


def _xg_body(q0_ref, q1_ref, h2_ref, d0_ref, d1_ref, b2_ref, wih_ref,
             bih_ref, bhh_ref, o_ref):
    dinv = _dinv(d0_ref, d1_ref)
    agg = q0_ref[...] + q1_ref[...] + h2_ref[...]
    t = jnp.maximum(agg * dinv + b2_ref[...], 0.0)
    o_ref[...] = (jnp.dot(t, wih_ref[...], preferred_element_type=jnp.float32)
                  + bih_ref[...] + bhh_ref[...])


def _lstm_body(xg_ref, whh_ref, wc_ref, bc_ref, o_ref, h_s, c_s):
    @pl.when(pl.program_id(0) == 0)
    def _init():
        h_s[...] = jnp.zeros_like(h_s)
        c_s[...] = jnp.zeros_like(c_s)

    def step(t, carry):
        h, c = carry
        gates = xg_ref[pl.ds(t, 1), :] + jnp.sum(
            whh_ref[...] * h.reshape(H, 1), axis=0, keepdims=True)
        i_g = jax.nn.sigmoid(gates[:, 0:H])
        f_g = jax.nn.sigmoid(gates[:, H:2 * H])
        g_g = jnp.tanh(gates[:, 2 * H:3 * H])
        o_g = jax.nn.sigmoid(gates[:, 3 * H:4 * H])
        c2 = f_g * c + i_g * g_g
        h2 = o_g * jnp.tanh(c2)
        return (h2, c2)

    h, c = lax.fori_loop(0, LBLK, step, (h_s[...], c_s[...]), unroll=4)
    h_s[...] = h
    c_s[...] = c

    @pl.when(pl.program_id(0) == LGRID - 1)
    def _final():
        o_ref[...] = jnp.dot(h, wc_ref[...],
                             preferred_element_type=jnp.float32) + bc_ref[...]


def _full(shape):
    return pl.BlockSpec(shape, lambda i: (0,) * len(shape))


def _rows(shape):
    return pl.BlockSpec(shape, lambda i: (i,) + (0,) * (len(shape) - 1))


_h1_call = pl.pallas_call(
    _h1_body,
    grid=(GRID,),
    in_specs=[_rows((BLK, D)), _full((D, H)), _rows((BLK, 16)), _rows((BLK, 16))],
    out_specs=_rows((BLK, H)),
    out_shape=jax.ShapeDtypeStruct((N_PAD, H), jnp.float32),
)

_h2_call = pl.pallas_call(
    _h2_body,
    grid=(GRID,),
    in_specs=[_rows((BLK, H)), _rows((BLK, H)), _rows((BLK, H)),
              _rows((BLK, 16)), _rows((BLK, 16)), _full((1, H)), _full((H, H))],
    out_specs=_rows((BLK, H)),
    out_shape=jax.ShapeDtypeStruct((N_PAD, H), jnp.float32),
)

_xg_call = pl.pallas_call(
    _xg_body,
    grid=(GRID,),
    in_specs=[_rows((BLK, H)), _rows((BLK, H)), _rows((BLK, H)),
              _rows((BLK, 16)), _rows((BLK, 16)), _full((1, H)),
              _full((H, 4 * H)), _full((1, 4 * H)), _full((1, 4 * H))],
    out_specs=_rows((BLK, 4 * H)),
    out_shape=jax.ShapeDtypeStruct((N_PAD, 4 * H), jnp.float32),
)

_lstm_call = pl.pallas_call(
    _lstm_body,
    grid=(LGRID,),
    in_specs=[_rows((LBLK, 4 * H)), _full((H, 4 * H)), _full((H, 128)),
              _full((1, 128))],
    out_specs=_full((1, 128)),
    out_shape=jax.ShapeDtypeStruct((1, 128), jnp.float32),
    scratch_shapes=[pltpu.VMEM((1, H), jnp.float32),
                    pltpu.VMEM((1, H), jnp.float32)],
)


def kernel(x, edge_index, timestamp, W1, b1, W2, b2, W_ih, W_hh, b_ih, b_hh,
           Wc, bc):
    pad = jnp.full((E_PAD - E,), DUMP, jnp.int32)
    src = jnp.concatenate([edge_index[0], pad])
    dst = jnp.concatenate([edge_index[1], pad]).reshape(NW, CPW, CHUNK)
    x_pad = jnp.pad(x, ((0, N_PAD - N), (0, 0)))
    z16 = jnp.zeros((RPS, 16), jnp.float32)
    z128 = jnp.zeros((RPS, D), jnp.float32)
    b1r = b1.reshape(1, H)
    b2r = b2.reshape(1, H)
    bihr = b_ih.reshape(1, 4 * H)
    bhhr = b_hh.reshape(1, 4 * H)
    whh_bf = W_hh
    wc_pad = jnp.pad(Wc, ((0, 0), (0, 128 - Wc.shape[1])))
    bc_pad = jnp.pad(bc, (0, 128 - bc.shape[0])).reshape(1, 128)

    degs = _deg_kernel(dst, z16)
    d0, d1 = degs[0], degs[1]
    h1p = _h1_call(x_pad, W1, d0, d1)
    agg1 = _agg_kernel(h1p, src, dst, z128)
    h2p = _h2_call(agg1[0], agg1[1], h1p, d0, d1, b1r, W2)
    agg2 = _agg_kernel(h2p, src, dst, z128)
    xg = _xg_call(agg2[0], agg2[1], h2p, d0, d1, b2r, W_ih, bihr, bhhr)
    out = _lstm_call(xg, whh_bf, wc_pad, bc_pad)
    return out[:, :bc.shape[0]]


# speedup vs baseline: 1.1304x; 1.0425x over previous
import jax
import jax.numpy as jnp
from jax import lax
from jax.experimental import pallas as pl
from jax.experimental.pallas import tpu as pltpu
from jax.experimental.pallas import tpu_sc as plsc

N = 10000
D = 128
H = 128
E = 320000

N_PAD = 10240
DUMP = N
NC, NS = 2, 16
NW = NC * NS
CHUNK = 128
CPW = 80
TCH = 2560
C0 = 120
C1 = 40
TCH_PAD = 2688
E_PAD = TCH * CHUNK
RPS = N_PAD // NS

BLK = 256
GRID = N_PAD // BLK
LBLK = 400
LGRID = N // LBLK


def _sc_mesh():
    return plsc.VectorSubcoreMesh(
        core_axis_name="c", subcore_axis_name="s", num_cores=NC, num_subcores=NS)


def _deg_body(dst_hbm, zeros_hbm, out_hbm, didx, ones_v, acc):
    c = lax.axis_index("c")
    s = lax.axis_index("s")
    wid = c * NS + s

    def put1(i, carry):
        ones_v[i, :] = jnp.ones((16,), jnp.float32)
        return carry

    lax.fori_loop(0, CHUNK, put1, 0)
    pltpu.sync_copy(dst_hbm.at[pl.ds(wid * CPW, CPW), :], didx)
    pltpu.sync_copy(zeros_hbm, acc.at[pl.ds(s * RPS, RPS), :])
    plsc.subcore_barrier()

    def step(k, carry):
        pltpu.sync_copy(ones_v, acc.at[didx.at[k]], add=True)
        return carry

    lax.fori_loop(0, CPW, step, 0)
    plsc.subcore_barrier()
    pltpu.sync_copy(acc.at[pl.ds(s * RPS, RPS), :],
                    out_hbm.at[c, pl.ds(s * RPS, RPS), :])


def _deg_kernel(*args):
    return pl.kernel(
        _deg_body,
        out_type=jax.ShapeDtypeStruct((NC, N_PAD, 16), jnp.float32),
        mesh=_sc_mesh(),
        scratch_types=[
            pltpu.VMEM((CPW, CHUNK), jnp.int32),
            pltpu.VMEM((CHUNK, 16), jnp.float32),
            pltpu.VMEM_SHARED((N_PAD, 16), jnp.float32),
        ],
    )(*args)


def _agg_body(table_hbm, src_hbm, dst_hbm, zeros_hbm, out_hbm,
              didx, sbuf0, sbuf1, rows0, rows1, acc,
              semi0, semi1, semr0, semr1):
    c = lax.axis_index("c")
    s = lax.axis_index("s")
    cb = jnp.where(c == 0, s * C0, NS * C0 + s * C1)
    ncpw = jnp.where(c == 0, C0, C1)
    base = cb * CHUNK
    pltpu.sync_copy(dst_hbm.at[pl.ds(cb, C0), :], didx)
    pltpu.sync_copy(zeros_hbm, acc.at[pl.ds(s * RPS, RPS), :])
    plsc.subcore_barrier()

    pltpu.sync_copy(src_hbm.at[pl.ds(base, CHUNK)], sbuf0)
    pltpu.async_copy(table_hbm.at[sbuf0], rows0, semr0)
    pltpu.async_copy(src_hbm.at[pl.ds(base + CHUNK, CHUNK)], sbuf1, semi1)

    def step(j, carry):
        a = 2 * j
        b = a + 1
        pltpu.make_async_copy(src_hbm.at[pl.ds(base + b * CHUNK, CHUNK)],
                              sbuf1, semi1).wait()
        pltpu.async_copy(table_hbm.at[sbuf1], rows1, semr1)
        pltpu.make_async_copy(table_hbm.at[sbuf0], rows0, semr0).wait()
        pltpu.sync_copy(rows0, acc.at[didx.at[a]], add=True)

        @pl.when(b + 1 < ncpw)
        def _prep_a():
            pltpu.async_copy(src_hbm.at[pl.ds(base + (b + 1) * CHUNK, CHUNK)],
                             sbuf0, semi0)

        pltpu.make_async_copy(table_hbm.at[sbuf1], rows1, semr1).wait()
        pltpu.sync_copy(rows1, acc.at[didx.at[b]], add=True)

        @pl.when(b + 1 < ncpw)
        def _gather_a():
            pltpu.make_async_copy(src_hbm.at[pl.ds(base + (b + 1) * CHUNK, CHUNK)],
                                  sbuf0, semi0).wait()
            pltpu.async_copy(table_hbm.at[sbuf0], rows0, semr0)

        @pl.when(b + 2 < ncpw)
        def _prep_b():
            pltpu.async_copy(src_hbm.at[pl.ds(base + (b + 2) * CHUNK, CHUNK)],
                             sbuf1, semi1)

        return carry

    lax.fori_loop(0, ncpw // 2, step, 0)
    plsc.subcore_barrier()
    pltpu.sync_copy(acc.at[pl.ds(s * RPS, RPS), :],
                    out_hbm.at[c, pl.ds(s * RPS, RPS), :])


def _agg_kernel(*args):
    return pl.kernel(
        _agg_body,
        out_type=jax.ShapeDtypeStruct((NC, N_PAD, D), jnp.float32),
        mesh=_sc_mesh(),
        scratch_types=[
            pltpu.VMEM((C0, CHUNK), jnp.int32),
            pltpu.VMEM((CHUNK,), jnp.int32),
            pltpu.VMEM((CHUNK,), jnp.int32),
            pltpu.VMEM((CHUNK, D), jnp.float32),
            pltpu.VMEM((CHUNK, D), jnp.float32),
            pltpu.VMEM_SHARED((N_PAD, D), jnp.float32),
            pltpu.SemaphoreType.DMA,
            pltpu.SemaphoreType.DMA,
            pltpu.SemaphoreType.DMA,
            pltpu.SemaphoreType.DMA,
        ],
    )(*args)


def _dinv(d0_ref, d1_ref):
    deg = d0_ref[:, 0:1] + d1_ref[:, 0:1] + 1.0
    return lax.rsqrt(deg)


def _h1_body(x_ref, w1_ref, d0_ref, d1_ref, o_ref):
    o_ref[...] = jnp.dot(x_ref[...], w1_ref[...],
                         preferred_element_type=jnp.float32) * _dinv(d0_ref, d1_ref)


def _h2_body(p0_ref, p1_ref, h1_ref, d0_ref, d1_ref, b1_ref, w2_ref, o_ref):
    dinv = _dinv(d0_ref, d1_ref)
    agg = p0_ref[...] + p1_ref[...] + h1_ref[...]
    t = jnp.maximum(agg * dinv + b1_ref[...], 0.0)
    h2 = jnp.dot(t, w2_ref[...], preferred_element_type=jnp.float32) * dinv
    row = pl.program_id(0) * BLK + lax.broadcasted_iota(jnp.int32, (BLK, 1), 0)
    o_ref[...] = jnp.where(row < N, h2, 0.0)


def _xg_body(q0_ref, q1_ref, h2_ref, d0_ref, d1_ref, b2_ref, wih_ref,
             bih_ref, bhh_ref, o_ref):
    dinv = _dinv(d0_ref, d1_ref)
    agg = q0_ref[...] + q1_ref[...] + h2_ref[...]
    t = jnp.maximum(agg * dinv + b2_ref[...], 0.0)
    o_ref[...] = (jnp.dot(t, wih_ref[...], preferred_element_type=jnp.float32)
                  + bih_ref[...] + bhh_ref[...])


def _lstm_body(xg_ref, whh_ref, wc_ref, bc_ref, o_ref, h_s, c_s):
    @pl.when(pl.program_id(0) == 0)
    def _init():
        h_s[...] = jnp.zeros_like(h_s)
        c_s[...] = jnp.zeros_like(c_s)

    def step(t, carry):
        h, c = carry
        gates = xg_ref[pl.ds(t, 1), :] + jnp.sum(
            whh_ref[...] * h.reshape(H, 1), axis=0, keepdims=True)
        i_g = jax.nn.sigmoid(gates[:, 0:H])
        f_g = jax.nn.sigmoid(gates[:, H:2 * H])
        g_g = jnp.tanh(gates[:, 2 * H:3 * H])
        o_g = jax.nn.sigmoid(gates[:, 3 * H:4 * H])
        c2 = f_g * c + i_g * g_g
        h2 = o_g * jnp.tanh(c2)
        return (h2, c2)

    h, c = lax.fori_loop(0, LBLK, step, (h_s[...], c_s[...]), unroll=4)
    h_s[...] = h
    c_s[...] = c

    @pl.when(pl.program_id(0) == LGRID - 1)
    def _final():
        o_ref[...] = jnp.dot(h, wc_ref[...],
                             preferred_element_type=jnp.float32) + bc_ref[...]


def _full(shape):
    return pl.BlockSpec(shape, lambda i: (0,) * len(shape))


def _rows(shape):
    return pl.BlockSpec(shape, lambda i: (i,) + (0,) * (len(shape) - 1))


_h1_call = pl.pallas_call(
    _h1_body,
    grid=(GRID,),
    in_specs=[_rows((BLK, D)), _full((D, H)), _rows((BLK, 16)), _rows((BLK, 16))],
    out_specs=_rows((BLK, H)),
    out_shape=jax.ShapeDtypeStruct((N_PAD, H), jnp.float32),
)

_h2_call = pl.pallas_call(
    _h2_body,
    grid=(GRID,),
    in_specs=[_rows((BLK, H)), _rows((BLK, H)), _rows((BLK, H)),
              _rows((BLK, 16)), _rows((BLK, 16)), _full((1, H)), _full((H, H))],
    out_specs=_rows((BLK, H)),
    out_shape=jax.ShapeDtypeStruct((N_PAD, H), jnp.float32),
)

_xg_call = pl.pallas_call(
    _xg_body,
    grid=(GRID,),
    in_specs=[_rows((BLK, H)), _rows((BLK, H)), _rows((BLK, H)),
              _rows((BLK, 16)), _rows((BLK, 16)), _full((1, H)),
              _full((H, 4 * H)), _full((1, 4 * H)), _full((1, 4 * H))],
    out_specs=_rows((BLK, 4 * H)),
    out_shape=jax.ShapeDtypeStruct((N_PAD, 4 * H), jnp.float32),
)

_lstm_call = pl.pallas_call(
    _lstm_body,
    grid=(LGRID,),
    in_specs=[_rows((LBLK, 4 * H)), _full((H, 4 * H)), _full((H, 128)),
              _full((1, 128))],
    out_specs=_full((1, 128)),
    out_shape=jax.ShapeDtypeStruct((1, 128), jnp.float32),
    scratch_shapes=[pltpu.VMEM((1, H), jnp.float32),
                    pltpu.VMEM((1, H), jnp.float32)],
)


def kernel(x, edge_index, timestamp, W1, b1, W2, b2, W_ih, W_hh, b_ih, b_hh,
           Wc, bc):
    pad = jnp.full((E_PAD - E,), DUMP, jnp.int32)
    src = jnp.concatenate([edge_index[0], pad])
    dpad = jnp.full((TCH_PAD * CHUNK - E,), DUMP, jnp.int32)
    dst = jnp.concatenate([edge_index[1], dpad]).reshape(TCH_PAD, CHUNK)
    x_pad = jnp.pad(x, ((0, N_PAD - N), (0, 0)))
    z16 = jnp.zeros((RPS, 16), jnp.float32)
    z128 = jnp.zeros((RPS, D), jnp.float32)
    b1r = b1.reshape(1, H)
    b2r = b2.reshape(1, H)
    bihr = b_ih.reshape(1, 4 * H)
    bhhr = b_hh.reshape(1, 4 * H)
    whh_bf = W_hh
    wc_pad = jnp.pad(Wc, ((0, 0), (0, 128 - Wc.shape[1])))
    bc_pad = jnp.pad(bc, (0, 128 - bc.shape[0])).reshape(1, 128)

    degs = _deg_kernel(dst, z16)
    d0, d1 = degs[0], degs[1]
    h1p = _h1_call(x_pad, W1, d0, d1)
    agg1 = _agg_kernel(h1p, src, dst, z128)
    h2p = _h2_call(agg1[0], agg1[1], h1p, d0, d1, b1r, W2)
    agg2 = _agg_kernel(h2p, src, dst, z128)
    xg = _xg_call(agg2[0], agg2[1], h2p, d0, d1, b2r, W_ih, bihr, bhhr)
    out = _lstm_call(xg, whh_bf, wc_pad, bc_pad)
    return out[:, :bc.shape[0]]
